# baseline (speedup 1.0000x reference)
import functools

import jax
import jax.numpy as jnp
from jax import lax
from jax.experimental import pallas as pl
from jax.experimental.pallas import tpu as pltpu
from jax.experimental.pallas import tpu_sc as plsc

D_MODEL = 1024
D_FF = 4096
N_EXPERTS = 9
N_ROUTED = 8
BM = 256
KFF = 4
FFB = D_FF // KFF
NC, NS = 2, 16
NW = NC * NS


def _gelu_exact(x):
    return 0.5 * x * (1.0 + lax.erf(x * 0.7071067811865476))


def _router_body(x_ref, ws_ref, bs_ref, routes_ref, scale_ref):
    logits = jnp.dot(x_ref[...], ws_ref[...],
                     preferred_element_type=jnp.float32) + bs_ref[...][None, :]
    m = jnp.max(logits, axis=1, keepdims=True)
    z = jnp.sum(jnp.exp(logits - m), axis=1, keepdims=True)
    p = 1.0 / z[:, 0]
    iota = lax.broadcasted_iota(jnp.int32, logits.shape, 1)
    routes = jnp.min(jnp.where(logits == m, iota, 127), axis=1)
    routes_ref[...] = routes
    scale_ref[...] = jnp.where(routes == 0, 0.0, p)


def _run_router(xf, Ws, bs):
    T = xf.shape[0]
    ws_pad = jnp.zeros((D_MODEL, 128), jnp.float32).at[:, :N_EXPERTS].set(Ws)
    bs_pad = jnp.full((128,), -1e30, jnp.float32).at[:N_EXPERTS].set(bs)
    return pl.pallas_call(
        _router_body,
        out_shape=(jax.ShapeDtypeStruct((T,), jnp.int32),
                   jax.ShapeDtypeStruct((T,), jnp.float32)),
    )(xf, ws_pad, bs_pad)


def _ff_body(be_ref, bv_ref, x_ref, w1_ref, b1_ref, w2_ref, b2_ref,
             u_ref, acc_ref):
    k = pl.program_id(0)
    m = pl.program_id(1)
    del be_ref
    valid = bv_ref[m] != 0

    @pl.when(valid)
    def _():
        h = jnp.dot(x_ref[...], w1_ref[0], preferred_element_type=jnp.float32)
        h = _gelu_exact(h + b1_ref[0, 0])
        contrib = jnp.dot(h, w2_ref[0], preferred_element_type=jnp.float32)

        @pl.when(k == 0)
        def _():
            acc_ref[m] = contrib + b2_ref[0]

        @pl.when(k > 0)
        def _():
            acc_ref[m] += contrib

        @pl.when(k == KFF - 1)
        def _():
            u_ref[...] = acc_ref[m]

    @pl.when(jnp.logical_and(jnp.logical_not(valid), k == KFF - 1))
    def _():
        u_ref[...] = jnp.zeros_like(u_ref)


def _run_ff(x, W1, b1, W2, b2, block_expert, block_valid):
    rows = x.shape[0]
    nblk = rows // BM
    grid_spec = pltpu.PrefetchScalarGridSpec(
        num_scalar_prefetch=2,
        grid=(KFF, nblk),
        in_specs=[
            pl.BlockSpec((BM, D_MODEL), lambda k, m, be, bv: (m, 0)),
            pl.BlockSpec((1, D_MODEL, FFB), lambda k, m, be, bv: (be[m], 0, k)),
            pl.BlockSpec((1, 1, 1, FFB), lambda k, m, be, bv: (be[m], k, 0, 0)),
            pl.BlockSpec((1, FFB, D_MODEL), lambda k, m, be, bv: (be[m], k, 0)),
            pl.BlockSpec((1, 1, D_MODEL), lambda k, m, be, bv: (be[m], 0, 0)),
        ],
        out_specs=pl.BlockSpec((BM, D_MODEL), lambda k, m, be, bv: (m, 0)),
        scratch_shapes=[pltpu.VMEM((nblk, BM, D_MODEL), jnp.float32)],
    )
    b1r = b1.reshape(N_EXPERTS, KFF, 1, FFB)
    b2r = b2.reshape(N_EXPERTS, 1, D_MODEL)
    return pl.pallas_call(
        _ff_body,
        grid_spec=grid_spec,
        out_shape=jax.ShapeDtypeStruct((rows, D_MODEL), jnp.float32),
    )(block_expert, block_valid, x, W1, b1r, W2, b2r)


def _make_sc_gather(P, b_per_w, chunk, ncol=D_MODEL, dtype=jnp.float32):
    mesh = plsc.VectorSubcoreMesh(core_axis_name="c", subcore_axis_name="s")
    nch = b_per_w // chunk

    @functools.partial(
        pl.kernel, mesh=mesh,
        out_type=jax.ShapeDtypeStruct((P, ncol), dtype),
        scratch_types=[
            pltpu.VMEM((b_per_w,), jnp.int32),
            pltpu.VMEM((3, chunk, ncol), dtype),
            pltpu.SemaphoreType.DMA,
            pltpu.SemaphoreType.DMA,
            pltpu.SemaphoreType.DMA,
            pltpu.SemaphoreType.DMA,
            pltpu.SemaphoreType.DMA,
            pltpu.SemaphoreType.DMA,
        ],
    )
    def gather_k(x_hbm, idx_hbm, out_hbm, idx_v, bufs,
                 g0, g1, g2, w0, w1, w2):
        gs = (g0, g1, g2)
        ws = (w0, w1, w2)
        wid = lax.axis_index("s") * NC + lax.axis_index("c")
        base = wid * b_per_w
        pltpu.sync_copy(idx_hbm.at[pl.ds(base, b_per_w)], idx_v)
        gcp = [None, None, None]
        wcp = [None, None, None]
        for ci in range(min(2, nch)):
            gcp[ci % 3] = pltpu.async_copy(
                x_hbm.at[idx_v.at[pl.ds(ci * chunk, chunk)]],
                bufs.at[ci % 3], gs[ci % 3])
        for ci in range(nch):
            b = ci % 3
            gcp[b].wait()
            wcp[b] = pltpu.async_copy(
                bufs.at[b], out_hbm.at[pl.ds(base + ci * chunk, chunk)],
                ws[b])
            nxt = ci + 2
            if nxt < nch:
                nb = nxt % 3
                if nxt >= 3:
                    wcp[nb].wait()
                gcp[nb] = pltpu.async_copy(
                    x_hbm.at[idx_v.at[pl.ds(nxt * chunk, chunk)]],
                    bufs.at[nb], gs[nb])
        for j in range(max(nch - 3, 0), nch):
            wcp[j % 3].wait()

    return gather_k


BB = 512


def _blend_body(u_ref, c_ref, s_ref, o_ref):
    s = jnp.reshape(s_ref[...], (BB, 1))
    c = c_ref[...]
    o_ref[...] = c + s * (u_ref[...] - c)


def _run_blend(u_g, common, scale):
    T = common.shape[0]
    return pl.pallas_call(
        _blend_body,
        grid=(T // BB,),
        in_specs=[
            pl.BlockSpec((BB, D_MODEL), lambda i: (i, 0)),
            pl.BlockSpec((BB, D_MODEL), lambda i: (i, 0)),
            pl.BlockSpec((BB,), lambda i: (i,)),
        ],
        out_specs=pl.BlockSpec((BB, D_MODEL), lambda i: (i, 0)),
        out_shape=jax.ShapeDtypeStruct((T, D_MODEL), jnp.float32),
    )(u_g, common, scale)


def _dispatch(routes, T, P):
    e_ids = jnp.arange(1, N_EXPERTS, dtype=routes.dtype)[:, None]
    masks = routes[None, :] == e_ids
    R = jnp.cumsum(masks.astype(jnp.int32), axis=1)
    c = R[:, -1]
    rank = jnp.sum(jnp.where(masks, R, 0), axis=0)
    blocks_per = (c + BM - 1) // BM
    cum_blocks = jnp.cumsum(blocks_per)
    pad_start = BM * (cum_blocks - blocks_per)
    nblk = P // BM

    bidx = jnp.arange(nblk, dtype=jnp.int32)
    grp_b = jnp.clip(jnp.searchsorted(cum_blocks, bidx, side="right"), 0, 7)
    n_used = cum_blocks[-1]
    block_valid = (bidx < n_used).astype(jnp.int32)
    last_e = grp_b[jnp.maximum(n_used - 1, 0)] + 1
    block_expert = jnp.where(block_valid == 1, grp_b + 1,
                             last_e).astype(jnp.int32)

    pad_start_full = jnp.concatenate(
        [jnp.zeros((1,), jnp.int32), pad_start.astype(jnp.int32)])
    pos = pad_start_full[routes] + rank - 1
    pos_scatter = jnp.where(routes == 0, P, pos)
    src_idx = jnp.zeros((P,), jnp.int32).at[pos_scatter].set(
        jnp.arange(T, dtype=jnp.int32), mode="drop")
    pos = jnp.where(routes == 0, 0, pos).astype(jnp.int32)
    return src_idx, pos, block_expert, block_valid


def kernel(x, W1, b1, W2, b2, Ws, bs):
    B, S, d = x.shape
    T = B * S
    P = T + N_ROUTED * BM
    xf = x.reshape(T, d)

    routes, scale = _run_router(xf, Ws, bs)
    src_idx, pos, block_expert, block_valid = _dispatch(routes, T, P)

    xg = _make_sc_gather(P, P // NW, 32)(xf, src_idx)

    common = _run_ff(xf, W1, b1, W2, b2,
                     jnp.zeros((T // BM,), jnp.int32),
                     jnp.ones((T // BM,), jnp.int32))
    u = _run_ff(xg, W1, b1, W2, b2, block_expert, block_valid)

    u_g = _make_sc_gather(T, T // NW, 32)(u, pos)
    out = _run_blend(u_g, common, scale)
    return out.reshape(B, S, d)


# speedup vs baseline: 1.0048x; 1.0048x over previous
import functools

import jax
import jax.numpy as jnp
from jax import lax
from jax.experimental import pallas as pl
from jax.experimental.pallas import tpu as pltpu
from jax.experimental.pallas import tpu_sc as plsc

D_MODEL = 1024
D_FF = 4096
N_EXPERTS = 9
N_ROUTED = 8
BM = 256
KFF = 4
FFB = D_FF // KFF
NC, NS = 2, 16
NW = NC * NS


def _gelu_exact(x):
    return 0.5 * x * (1.0 + lax.erf(x * 0.7071067811865476))


def _router_body(x_ref, ws_ref, bs_ref, routes_ref, scale_ref):
    logits = jnp.dot(x_ref[...], ws_ref[...],
                     preferred_element_type=jnp.float32) + bs_ref[...][None, :]
    m = jnp.max(logits, axis=1, keepdims=True)
    z = jnp.sum(jnp.exp(logits - m), axis=1, keepdims=True)
    p = 1.0 / z[:, 0]
    iota = lax.broadcasted_iota(jnp.int32, logits.shape, 1)
    routes = jnp.min(jnp.where(logits == m, iota, 127), axis=1)
    routes_ref[...] = routes
    scale_ref[...] = jnp.where(routes == 0, 0.0, p)


def _run_router(xf, Ws, bs):
    T = xf.shape[0]
    ws_pad = jnp.zeros((D_MODEL, 128), jnp.float32).at[:, :N_EXPERTS].set(Ws)
    bs_pad = jnp.full((128,), -1e30, jnp.float32).at[:N_EXPERTS].set(bs)
    return pl.pallas_call(
        _router_body,
        out_shape=(jax.ShapeDtypeStruct((T,), jnp.int32),
                   jax.ShapeDtypeStruct((T,), jnp.float32)),
    )(xf, ws_pad, bs_pad)


def _ff_body(be_ref, bv_ref, x_ref, w1_ref, b1_ref, w2_ref, b2_ref,
             u_ref, acc_ref):
    k = pl.program_id(0)
    m = pl.program_id(1)
    del be_ref
    valid = bv_ref[m] != 0

    @pl.when(valid)
    def _():
        h = jnp.dot(x_ref[...], w1_ref[0], preferred_element_type=jnp.float32)
        h = _gelu_exact(h + b1_ref[0, 0])
        contrib = jnp.dot(h, w2_ref[0], preferred_element_type=jnp.float32)

        @pl.when(k == 0)
        def _():
            acc_ref[m] = contrib + b2_ref[0]

        @pl.when(k > 0)
        def _():
            acc_ref[m] += contrib

        @pl.when(k == KFF - 1)
        def _():
            u_ref[...] = acc_ref[m]

    @pl.when(jnp.logical_and(jnp.logical_not(valid), k == KFF - 1))
    def _():
        u_ref[...] = jnp.zeros_like(u_ref)


def _run_ff(x, W1, b1, W2, b2, block_expert, block_valid):
    rows = x.shape[0]
    nblk = rows // BM
    grid_spec = pltpu.PrefetchScalarGridSpec(
        num_scalar_prefetch=2,
        grid=(KFF, nblk),
        in_specs=[
            pl.BlockSpec((BM, D_MODEL), lambda k, m, be, bv: (m, 0)),
            pl.BlockSpec((1, D_MODEL, FFB), lambda k, m, be, bv: (be[m], 0, k)),
            pl.BlockSpec((1, 1, 1, FFB), lambda k, m, be, bv: (be[m], k, 0, 0)),
            pl.BlockSpec((1, FFB, D_MODEL), lambda k, m, be, bv: (be[m], k, 0)),
            pl.BlockSpec((1, 1, D_MODEL), lambda k, m, be, bv: (be[m], 0, 0)),
        ],
        out_specs=pl.BlockSpec((BM, D_MODEL), lambda k, m, be, bv: (m, 0)),
        scratch_shapes=[pltpu.VMEM((nblk, BM, D_MODEL), jnp.float32)],
    )
    b1r = b1.reshape(N_EXPERTS, KFF, 1, FFB)
    b2r = b2.reshape(N_EXPERTS, 1, D_MODEL)
    return pl.pallas_call(
        _ff_body,
        grid_spec=grid_spec,
        out_shape=jax.ShapeDtypeStruct((rows, D_MODEL), jnp.float32),
    )(block_expert, block_valid, x, W1, b1r, W2, b2r)


def _make_sc_gather(P, b_per_w, chunk, ncol=D_MODEL, dtype=jnp.float32):
    mesh = plsc.VectorSubcoreMesh(core_axis_name="c", subcore_axis_name="s")
    nch = b_per_w // chunk

    @functools.partial(
        pl.kernel, mesh=mesh,
        out_type=jax.ShapeDtypeStruct((P, ncol), dtype),
        scratch_types=[
            pltpu.VMEM((b_per_w,), jnp.int32),
            pltpu.VMEM((2, chunk, ncol), dtype),
            pltpu.SemaphoreType.DMA,
            pltpu.SemaphoreType.DMA,
        ],
    )
    def gather_k(x_hbm, idx_hbm, out_hbm, idx_v, bufs, sem0, sem1):
        wid = lax.axis_index("s") * NC + lax.axis_index("c")
        base = wid * b_per_w
        sems = (sem0, sem1)
        pltpu.sync_copy(idx_hbm.at[pl.ds(base, b_per_w)], idx_v)
        cps = [None, None]
        cps[0] = pltpu.async_copy(
            x_hbm.at[idx_v.at[pl.ds(0, chunk)]], bufs.at[0], sems[0])
        for ci in range(1, nch):
            b = ci % 2
            cps[b] = pltpu.async_copy(
                x_hbm.at[idx_v.at[pl.ds(ci * chunk, chunk)]], bufs.at[b],
                sems[b])
            cps[1 - b].wait()
            pltpu.sync_copy(bufs.at[1 - b],
                            out_hbm.at[pl.ds(base + (ci - 1) * chunk, chunk)])
        last = (nch - 1) % 2
        cps[last].wait()
        pltpu.sync_copy(bufs.at[last],
                        out_hbm.at[pl.ds(base + (nch - 1) * chunk, chunk)])

    return gather_k


BB = 512


def _blend_body(u_ref, c_ref, s_ref, o_ref):
    s = jnp.reshape(s_ref[...], (BB, 1))
    c = c_ref[...]
    o_ref[...] = c + s * (u_ref[...] - c)


def _run_blend(u_g, common, scale):
    T = common.shape[0]
    return pl.pallas_call(
        _blend_body,
        grid=(T // BB,),
        in_specs=[
            pl.BlockSpec((BB, D_MODEL), lambda i: (i, 0)),
            pl.BlockSpec((BB, D_MODEL), lambda i: (i, 0)),
            pl.BlockSpec((BB,), lambda i: (i,)),
        ],
        out_specs=pl.BlockSpec((BB, D_MODEL), lambda i: (i, 0)),
        out_shape=jax.ShapeDtypeStruct((T, D_MODEL), jnp.float32),
    )(u_g, common, scale)


def _dispatch(routes, T, P):
    e_ids = jnp.arange(1, N_EXPERTS, dtype=routes.dtype)[:, None]
    masks = routes[None, :] == e_ids
    R = jnp.cumsum(masks.astype(jnp.int32), axis=1)
    c = R[:, -1]
    rank = jnp.sum(jnp.where(masks, R, 0), axis=0)
    blocks_per = (c + BM - 1) // BM
    cum_blocks = jnp.cumsum(blocks_per)
    pad_start = BM * (cum_blocks - blocks_per)
    nblk = P // BM

    bidx = jnp.arange(nblk, dtype=jnp.int32)
    grp_b = jnp.clip(jnp.searchsorted(cum_blocks, bidx, side="right"), 0, 7)
    n_used = cum_blocks[-1]
    block_valid = (bidx < n_used).astype(jnp.int32)
    last_e = grp_b[jnp.maximum(n_used - 1, 0)] + 1
    block_expert = jnp.where(block_valid == 1, grp_b + 1,
                             last_e).astype(jnp.int32)

    pad_start_full = jnp.concatenate(
        [jnp.zeros((1,), jnp.int32), pad_start.astype(jnp.int32)])
    pos = pad_start_full[routes] + rank - 1
    pos_scatter = jnp.where(routes == 0, P, pos)
    src_idx = jnp.zeros((P,), jnp.int32).at[pos_scatter].set(
        jnp.arange(T, dtype=jnp.int32), mode="drop")
    pos = jnp.where(routes == 0, 0, pos).astype(jnp.int32)
    return src_idx, pos, block_expert, block_valid


def kernel(x, W1, b1, W2, b2, Ws, bs):
    B, S, d = x.shape
    T = B * S
    P = T + N_ROUTED * BM
    xf = x.reshape(T, d)

    routes, scale = _run_router(xf, Ws, bs)
    src_idx, pos, block_expert, block_valid = _dispatch(routes, T, P)

    xg = _make_sc_gather(P, P // NW, 48)(xf, src_idx)

    common = _run_ff(xf, W1, b1, W2, b2,
                     jnp.zeros((T // BM,), jnp.int32),
                     jnp.ones((T // BM,), jnp.int32))
    u = _run_ff(xg, W1, b1, W2, b2, block_expert, block_valid)

    u_g = _make_sc_gather(T, T // NW, 32)(u, pos)
    out = _run_blend(u_g, common, scale)
    return out.reshape(B, S, d)
